# baseline (speedup 1.0000x reference)
import functools

import jax
import jax.numpy as jnp
from jax import lax
from jax.experimental import pallas as pl
from jax.experimental.pallas import tpu as pltpu
from jax.experimental.pallas import tpu_sc as plsc

_BD = 20.0
_BM = 0.1
_NS = 1000
_TAB_PAD = 1024

_NC = 2
_NSUB = 16
_LANES = 16
_NW = _NC * _NSUB

_B = 512
_D = 3 * 128 * 128
_ROWS = 32
_TW = 128


def _coeff_table():
    ts = jnp.linspace(0.0, 1.0, _NS, dtype=jnp.float32)
    betas = (_BM + (_BD - _BM) * ts) / _NS
    alphas = 1.0 - betas
    ac = jnp.cumprod(alphas, axis=0)
    sqrt_recip = jnp.sqrt(1.0 / ac)
    sqrt_recipm1 = jnp.sqrt(1.0 / ac - 1.0)
    tab_a = jnp.tile(sqrt_recip[:, None], (1, _TW))
    tab_b = jnp.tile(sqrt_recipm1[:, None], (1, _TW))
    return tab_a, tab_b


_SC_CORES = 1
_SC_CHUNK = _B // (_SC_CORES * _NSUB)


def _sc_gather_body(ta_hbm, tb_hbm, idx_hbm, oa_hbm, ob_hbm,
                    idx_v, ra_v, rb_v, sem_a, sem_b):
    wid = lax.axis_index("s") * _SC_CORES + lax.axis_index("c")
    base = wid * _SC_CHUNK
    pltpu.sync_copy(idx_hbm.at[pl.ds(base, _SC_CHUNK)], idx_v)
    cp_a = pltpu.async_copy(ta_hbm.at[idx_v], ra_v, sem_a)
    cp_b = pltpu.async_copy(tb_hbm.at[idx_v], rb_v, sem_b)
    cp_a.wait()
    cp_b.wait()
    pltpu.sync_copy(ra_v, oa_hbm.at[pl.ds(base, _SC_CHUNK)])
    pltpu.sync_copy(rb_v, ob_hbm.at[pl.ds(base, _SC_CHUNK)])


@functools.lru_cache(maxsize=1)
def _sc_gather():
    return pl.kernel(
        _sc_gather_body,
        out_type=(
            jax.ShapeDtypeStruct((_B, _TW), jnp.float32),
            jax.ShapeDtypeStruct((_B, _TW), jnp.float32),
        ),
        mesh=plsc.VectorSubcoreMesh(
            core_axis_name="c", subcore_axis_name="s", num_cores=_SC_CORES),
        scratch_types=[
            pltpu.VMEM((_SC_CHUNK,), jnp.int32),
            pltpu.VMEM((_SC_CHUNK, _TW), jnp.float32),
            pltpu.VMEM((_SC_CHUNK, _TW), jnp.float32),
            pltpu.SemaphoreType.DMA,
            pltpu.SemaphoreType.DMA,
        ],
    )


def _tc_fma_body(ca_ref, cb_ref, x_ref, n_ref, o_ref):
    r0 = pl.program_id(0) * _ROWS
    for r in range(_ROWS):
        a_row = ca_ref[r0 + r]
        b_row = cb_ref[r0 + r]
        o_ref[r] = a_row * x_ref[r] - b_row * n_ref[r]


def _tc_fma(coeff_a, coeff_b, x4, n4):
    grid = (_B // _ROWS,)
    coeff_spec = pl.BlockSpec((_B, _TW), lambda r: (0, 0))
    row_spec = pl.BlockSpec((_ROWS, 3, 128, 128), lambda r: (r, 0, 0, 0))
    return pl.pallas_call(
        _tc_fma_body,
        grid=grid,
        in_specs=[coeff_spec, coeff_spec, row_spec, row_spec],
        out_specs=row_spec,
        out_shape=jax.ShapeDtypeStruct((_B, 3, 128, 128), jnp.float32),
    )(coeff_a, coeff_b, x4, n4)


def kernel(x_i, noise, i):
    tab_a, tab_b = _coeff_table()
    coeff_a, coeff_b = _sc_gather()(tab_a, tab_b, i.astype(jnp.int32))
    return _tc_fma(coeff_a, coeff_b, x_i, noise)


# speedup vs baseline: 1.0006x; 1.0006x over previous
import functools

import jax
import jax.numpy as jnp
from jax import lax
from jax.experimental import pallas as pl
from jax.experimental.pallas import tpu as pltpu
from jax.experimental.pallas import tpu_sc as plsc

_BD = 20.0
_BM = 0.1
_NS = 1000

_B = 512
_ROWS = 32
_TW = 128

_NSUB = 16
_SC_CORES = 1
_SC_CHUNK = _B // (_SC_CORES * _NSUB)


def _coeff_tables():
    ts = jnp.linspace(0.0, 1.0, _NS, dtype=jnp.float32)
    betas = (_BM + (_BD - _BM) * ts) / _NS
    alphas = 1.0 - betas
    ac = jnp.cumprod(alphas, axis=0)
    sqrt_recip = jnp.sqrt(1.0 / ac)
    sqrt_recipm1 = jnp.sqrt(1.0 / ac - 1.0)
    tab_a = jnp.tile(sqrt_recip[:, None], (1, _TW))
    tab_b = jnp.tile(sqrt_recipm1[:, None], (1, _TW))
    return tab_a, tab_b


def _sc_gather_body(ta_hbm, tb_hbm, idx_hbm, oa_hbm, ob_hbm,
                    idx_v, ra_v, rb_v, sem_a, sem_b):
    wid = lax.axis_index("s") * _SC_CORES + lax.axis_index("c")
    base = wid * _SC_CHUNK
    pltpu.sync_copy(idx_hbm.at[pl.ds(base, _SC_CHUNK)], idx_v)
    cp_a = pltpu.async_copy(ta_hbm.at[idx_v], ra_v, sem_a)
    cp_b = pltpu.async_copy(tb_hbm.at[idx_v], rb_v, sem_b)
    cp_a.wait()
    cp_b.wait()
    pltpu.sync_copy(ra_v, oa_hbm.at[pl.ds(base, _SC_CHUNK)])
    pltpu.sync_copy(rb_v, ob_hbm.at[pl.ds(base, _SC_CHUNK)])


@functools.lru_cache(maxsize=1)
def _sc_gather():
    return pl.kernel(
        _sc_gather_body,
        out_type=(
            jax.ShapeDtypeStruct((_B, _TW), jnp.float32),
            jax.ShapeDtypeStruct((_B, _TW), jnp.float32),
        ),
        mesh=plsc.VectorSubcoreMesh(
            core_axis_name="c", subcore_axis_name="s", num_cores=_SC_CORES),
        scratch_types=[
            pltpu.VMEM((_SC_CHUNK,), jnp.int32),
            pltpu.VMEM((_SC_CHUNK, _TW), jnp.float32),
            pltpu.VMEM((_SC_CHUNK, _TW), jnp.float32),
            pltpu.SemaphoreType.DMA,
            pltpu.SemaphoreType.DMA,
        ],
    )


def _tc_fma_body(ca_ref, cb_ref, x_ref, n_ref, o_ref):
    r0 = pl.program_id(0) * _ROWS
    for r in range(_ROWS):
        a_row = ca_ref[r0 + r]
        b_row = cb_ref[r0 + r]
        o_ref[r] = a_row * x_ref[r] - b_row * n_ref[r]


def _tc_fma(coeff_a, coeff_b, x4, n4):
    grid = (_B // _ROWS,)
    coeff_spec = pl.BlockSpec((_B, _TW), lambda r: (0, 0))
    row_spec = pl.BlockSpec((_ROWS, 3, 128, 128), lambda r: (r, 0, 0, 0))
    return pl.pallas_call(
        _tc_fma_body,
        grid=grid,
        in_specs=[coeff_spec, coeff_spec, row_spec, row_spec],
        out_specs=row_spec,
        out_shape=jax.ShapeDtypeStruct((_B, 3, 128, 128), jnp.float32),
    )(coeff_a, coeff_b, x4, n4)


def kernel(x_i, noise, i):
    tab_a, tab_b = _coeff_tables()
    coeff_a, coeff_b = _sc_gather()(tab_a, tab_b, i.astype(jnp.int32))
    return _tc_fma(coeff_a, coeff_b, x_i, noise)


# speedup vs baseline: 1.0024x; 1.0018x over previous
import functools

import jax
import jax.numpy as jnp
from jax import lax
from jax.experimental import pallas as pl
from jax.experimental.pallas import tpu as pltpu
from jax.experimental.pallas import tpu_sc as plsc

_BD = 20.0
_BM = 0.1
_NS = 1000

_B = 512
_ROWS = 32
_TW = 128

_NSUB = 16
_SC_CORES = 1
_SC_CHUNK = _B // (_SC_CORES * _NSUB)


def _coeff_tables():
    ts = jnp.linspace(0.0, 1.0, _NS, dtype=jnp.float32)
    betas = (_BM + (_BD - _BM) * ts) / _NS
    alphas = 1.0 - betas
    ac = jnp.cumprod(alphas, axis=0)
    sqrt_recip = jnp.sqrt(1.0 / ac)
    sqrt_recipm1 = jnp.sqrt(1.0 / ac - 1.0)
    tab_a = jnp.tile(sqrt_recip[:, None], (1, _TW))
    tab_b = jnp.tile(sqrt_recipm1[:, None], (1, _TW))
    return tab_a, tab_b


def _sc_gather_body(ta_hbm, tb_hbm, idx_hbm, oa_hbm, ob_hbm,
                    idx_v, ra_v, rb_v, sem_a, sem_b):
    wid = lax.axis_index("s") * _SC_CORES + lax.axis_index("c")
    base = wid * _SC_CHUNK
    pltpu.sync_copy(idx_hbm.at[pl.ds(base, _SC_CHUNK)], idx_v)
    cp_a = pltpu.async_copy(ta_hbm.at[idx_v], ra_v, sem_a)
    cp_b = pltpu.async_copy(tb_hbm.at[idx_v], rb_v, sem_b)
    cp_a.wait()
    wr_a = pltpu.async_copy(ra_v, oa_hbm.at[pl.ds(base, _SC_CHUNK)], sem_a)
    cp_b.wait()
    wr_b = pltpu.async_copy(rb_v, ob_hbm.at[pl.ds(base, _SC_CHUNK)], sem_b)
    wr_a.wait()
    wr_b.wait()


@functools.lru_cache(maxsize=1)
def _sc_gather():
    return pl.kernel(
        _sc_gather_body,
        out_type=(
            jax.ShapeDtypeStruct((_B, _TW), jnp.float32),
            jax.ShapeDtypeStruct((_B, _TW), jnp.float32),
        ),
        mesh=plsc.VectorSubcoreMesh(
            core_axis_name="c", subcore_axis_name="s", num_cores=_SC_CORES),
        scratch_types=[
            pltpu.VMEM((_SC_CHUNK,), jnp.int32),
            pltpu.VMEM((_SC_CHUNK, _TW), jnp.float32),
            pltpu.VMEM((_SC_CHUNK, _TW), jnp.float32),
            pltpu.SemaphoreType.DMA,
            pltpu.SemaphoreType.DMA,
        ],
    )


def _tc_fma_body(ca_ref, cb_ref, x_ref, n_ref, o_ref):
    r0 = pl.program_id(0) * _ROWS
    for r in range(_ROWS):
        a_row = ca_ref[r0 + r]
        b_row = cb_ref[r0 + r]
        o_ref[r] = a_row * x_ref[r] - b_row * n_ref[r]


def _tc_fma(coeff_a, coeff_b, x4, n4):
    grid = (_B // _ROWS,)
    coeff_spec = pl.BlockSpec((_B, _TW), lambda r: (0, 0))
    row_spec = pl.BlockSpec((_ROWS, 3, 128, 128), lambda r: (r, 0, 0, 0))
    return pl.pallas_call(
        _tc_fma_body,
        grid=grid,
        in_specs=[coeff_spec, coeff_spec, row_spec, row_spec],
        out_specs=row_spec,
        out_shape=jax.ShapeDtypeStruct((_B, 3, 128, 128), jnp.float32),
    )(coeff_a, coeff_b, x4, n4)


def kernel(x_i, noise, i):
    tab_a, tab_b = _coeff_tables()
    coeff_a, coeff_b = _sc_gather()(tab_a, tab_b, i.astype(jnp.int32))
    return _tc_fma(coeff_a, coeff_b, x_i, noise)
